# baseline (speedup 1.0000x reference)
import functools

import jax
import jax.numpy as jnp
from jax import lax
from jax.experimental import pallas as pl
from jax.experimental.pallas import tpu as pltpu
from jax.experimental.pallas import tpu_sc as plsc

NUM_HEAD = 100
EMBED_DIM = 256
BATCH = 16384

_info = plsc.get_sparse_core_info()
_NC, _NS, _NL = _info.num_cores, _info.num_subcores, _info.num_lanes
_NW = _NC * _NS
_B_PER_W = BATCH // _NW
_CHUNK = 128
_NCHUNK = _B_PER_W // _CHUNK
_NBUF = 3

_mesh = plsc.VectorSubcoreMesh(core_axis_name="c", subcore_axis_name="s")


@functools.partial(
    pl.kernel,
    mesh=_mesh,
    out_type=jax.ShapeDtypeStruct((BATCH, 1, EMBED_DIM), jnp.float32),
    scratch_types=[
        pltpu.VMEM((NUM_HEAD, EMBED_DIM), jnp.float32),
        pltpu.VMEM((_B_PER_W,), jnp.int32),
    ]
    + [pltpu.VMEM((_CHUNK, EMBED_DIM), jnp.float32)] * _NBUF
    + [pltpu.SemaphoreType.DMA] * _NBUF,
)
def _gather_kernel(table_hbm, idx_hbm, out_hbm, table_v, idx_v, *scratch):
    bufs = scratch[:_NBUF]
    wsems = scratch[_NBUF:]
    wid = lax.axis_index("s") * _NC + lax.axis_index("c")
    base = wid * _B_PER_W

    pltpu.sync_copy(idx_hbm.at[pl.ds(base, _B_PER_W)], idx_v)
    pltpu.sync_copy(table_hbm, table_v)

    w = [None] * _NBUF
    for c in range(_NCHUNK):
        b = c % _NBUF
        if w[b] is not None:
            w[b].wait()
        buf = bufs[b]

        def body(g, _, c=c, buf=buf):
            lblv = idx_v[pl.ds(c * _CHUNK + g * _NL, _NL)]
            for k in range(_NL):
                lbl = lblv[k]
                r = g * _NL + k
                for j in range(EMBED_DIM // _NL):
                    buf[r, pl.ds(j * _NL, _NL)] = table_v[lbl, pl.ds(j * _NL, _NL)]
            return 0

        lax.fori_loop(0, _CHUNK // _NL, body, 0)
        w[b] = pltpu.async_copy(
            buf, out_hbm.at[pl.ds(base + c * _CHUNK, _CHUNK), 0], wsems[b]
        )
    for c in range(max(0, _NCHUNK - _NBUF), _NCHUNK):
        w[c % _NBUF].wait()


def kernel(params, labels, head_table):
    del params
    return _gather_kernel(head_table, labels)


# speedup vs baseline: 1.7639x; 1.7639x over previous
import functools

import jax
import jax.numpy as jnp
from jax import lax
from jax.experimental import pallas as pl
from jax.experimental.pallas import tpu as pltpu
from jax.experimental.pallas import tpu_sc as plsc

NUM_HEAD = 100
EMBED_DIM = 256
BATCH = 16384

_info = plsc.get_sparse_core_info()
_NC, _NS = _info.num_cores, _info.num_subcores
_NW = _NC * _NS
_B_PER_W = BATCH // _NW
_CHUNK = 128

_mesh = plsc.VectorSubcoreMesh(core_axis_name="c", subcore_axis_name="s")


@functools.partial(
    pl.kernel,
    mesh=_mesh,
    out_type=jax.ShapeDtypeStruct((BATCH, 1, EMBED_DIM), jnp.float32),
    scratch_types=[
        pltpu.VMEM((NUM_HEAD * EMBED_DIM,), jnp.float32),
        pltpu.VMEM((_B_PER_W,), jnp.int32),
        pltpu.VMEM((_CHUNK, EMBED_DIM), jnp.float32),
        pltpu.SemaphoreType.DMA,
    ],
)
def _gather_kernel(table_hbm, idx_hbm, out_hbm, table_v, idx_v, dummy_v, sem):
    wid = lax.axis_index("s") * _NC + lax.axis_index("c")
    base = wid * _B_PER_W

    pltpu.sync_copy(idx_hbm.at[pl.ds(base, _B_PER_W)], idx_v)
    pltpu.sync_copy(table_hbm, table_v)

    _NL = 16

    def body(g, _):
        lblv = idx_v[pl.ds(g * _NL, _NL)] * EMBED_DIM
        for k in range(_NL):
            pltpu.async_copy(
                table_v.at[pl.ds(pl.multiple_of(lblv[k], EMBED_DIM), EMBED_DIM)],
                out_hbm.at[base + g * _NL + k, 0],
                sem,
            )
        return 0

    lax.fori_loop(0, _B_PER_W // _NL, body, 0)
    for i in range(_B_PER_W // _CHUNK):
        pltpu.make_async_copy(
            out_hbm.at[pl.ds(base + i * _CHUNK, _CHUNK), 0], dummy_v, sem
        ).wait()


def kernel(params, labels, head_table):
    del params
    return _gather_kernel(head_table.reshape(-1), labels)


# speedup vs baseline: 1.7781x; 1.0081x over previous
import functools

import jax
import jax.numpy as jnp
from jax import lax
from jax.experimental import pallas as pl
from jax.experimental.pallas import tpu as pltpu
from jax.experimental.pallas import tpu_sc as plsc

NUM_HEAD = 100
EMBED_DIM = 256
BATCH = 16384

_info = plsc.get_sparse_core_info()
_NC, _NS = _info.num_cores, _info.num_subcores
_NW = _NC * _NS
_B_PER_W = BATCH // _NW
_CHUNK = 128

_mesh = plsc.VectorSubcoreMesh(core_axis_name="c", subcore_axis_name="s")


@functools.partial(
    pl.kernel,
    mesh=_mesh,
    out_type=jax.ShapeDtypeStruct((BATCH, 1, EMBED_DIM), jnp.float32),
    scratch_types=[
        pltpu.VMEM((NUM_HEAD, EMBED_DIM), jnp.float32),
        pltpu.VMEM((_B_PER_W,), jnp.int32),
        pltpu.VMEM((_CHUNK, EMBED_DIM), jnp.float32),
        pltpu.SemaphoreType.DMA,
    ],
)
def _gather_kernel(table_hbm, idx_hbm, out_hbm, table_v, idx_v, dummy_v, sem):
    wid = lax.axis_index("s") * _NC + lax.axis_index("c")
    base = wid * _B_PER_W

    pltpu.sync_copy(idx_hbm.at[pl.ds(base, _B_PER_W)], idx_v)
    pltpu.sync_copy(table_hbm, table_v)

    _NL = 16

    def body(g, _):
        lblv = idx_v[pl.ds(g * _NL, _NL)]
        for k in range(_NL):
            pltpu.async_copy(
                table_v.at[lblv[k]],
                out_hbm.at[base + g * _NL + k, 0],
                sem,
            )
        return 0

    lax.fori_loop(0, _B_PER_W // _NL, body, 0)
    for i in range(_B_PER_W // _CHUNK):
        pltpu.make_async_copy(
            out_hbm.at[pl.ds(base + i * _CHUNK, _CHUNK), 0], dummy_v, sem
        ).wait()


def kernel(params, labels, head_table):
    del params
    return _gather_kernel(head_table, labels)
